# baseline (speedup 1.0000x reference)
import functools

import jax
import jax.numpy as jnp
from jax import lax
from jax.experimental import pallas as pl
from jax.experimental.pallas import tpu as pltpu
from jax.experimental.pallas import tpu_sc as plsc

N = 16384
H = N // 2
D = 64
K = 512
W = 128
BLK = 2048
GRID_H = H // BLK

NC = 2
NS = 16
NW = NC * NS
RPW = H // NW
CHUNK = 128
NCHUNK = RPW // CHUNK

_IDXMASK = 511


def _assign_body(z_ref, c_ref, assign_ref, dc_ref, zaug_ref, caug_ref, dcv_ref):
    i = pl.program_id(0)

    @pl.when(i == 0)
    def _init():
        c = c_ref[...]
        cc = jnp.sum(c * c, axis=1, keepdims=True)
        caug_ref[...] = jnp.concatenate(
            [c, jnp.ones((K, 1), jnp.float32), cc], axis=1)
        dcv_ref[...] = jnp.zeros_like(dcv_ref)

    z = z_ref[...]
    zz = jnp.sum(z * z, axis=1, keepdims=True)
    ones = jnp.ones((BLK, 1), jnp.float32)
    zaug_ref[...] = jnp.concatenate(
        [z, ones, jnp.zeros((BLK, W - D - 1), jnp.float32)], axis=1)
    d2 = jnp.maximum(jax.lax.dot_general(
        jnp.concatenate([-2.0 * z, zz, ones], axis=1), caug_ref[...],
        dimension_numbers=(((1,), (1,)), ((), ())),
        preferred_element_type=jnp.float32,
    ), 0.0)

    iota = jax.lax.broadcasted_iota(jnp.int32, (BLK, K), 1)
    key = (jax.lax.bitcast_convert_type(d2, jnp.int32) & ~_IDXMASK) | iota
    kmin = jnp.min(key, axis=1, keepdims=True)
    assign_ref[...] = kmin & _IDXMASK
    m = jax.lax.bitcast_convert_type(kmin & ~_IDXMASK, jnp.float32)
    dcv_ref[...] += jnp.sqrt(m + 1e-12)

    @pl.when(i == GRID_H - 1)
    def _finish():
        dc_ref[...] = jnp.sum(dcv_ref[...]).reshape(1, 1)


def _assign_stage(z_half, centers):
    return pl.pallas_call(
        _assign_body,
        grid=(GRID_H,),
        in_specs=[
            pl.BlockSpec((BLK, D), lambda i: (i, 0)),
            pl.BlockSpec((K, D), lambda i: (0, 0)),
        ],
        out_specs=[
            pl.BlockSpec((BLK, 1), lambda i: (i, 0)),
            pl.BlockSpec((1, 1), lambda i: (0, 0)),
            pl.BlockSpec((BLK, W), lambda i: (i, 0)),
        ],
        out_shape=[
            jax.ShapeDtypeStruct((H, 1), jnp.int32),
            jax.ShapeDtypeStruct((1, 1), jnp.float32),
            jax.ShapeDtypeStruct((H, W), jnp.float32),
        ],
        scratch_shapes=[
            pltpu.VMEM((K, D + 2), jnp.float32),
            pltpu.VMEM((BLK, 1), jnp.float32),
        ],
    )(z_half, centers)


def _segsum_kernel(zaug_hbm, idx_hbm, zeros_hbm, sums_hbm,
                   idx_v, zrows_v, acc_s):
    c = lax.axis_index("c")
    s = lax.axis_index("s")
    wid = s * NC + c

    pltpu.sync_copy(idx_hbm.at[wid], idx_v)

    @pl.when(s == 0)
    def _zero():
        pltpu.sync_copy(zeros_hbm, acc_s)

    plsc.subcore_barrier()

    for j in range(NCHUNK):
        pltpu.sync_copy(zaug_hbm.at[pl.ds(wid * RPW + j * CHUNK, CHUNK)],
                        zrows_v)
        pltpu.sync_copy(zrows_v, acc_s.at[idx_v.at[j]], add=True)

    plsc.subcore_barrier()

    @pl.when(s == 0)
    def _flush():
        pltpu.sync_copy(acc_s, sums_hbm.at[c])


def _segsum_stage(zaug_half, assign_half):
    idx3 = assign_half.reshape(NW, NCHUNK, CHUNK)
    run = pl.kernel(
        _segsum_kernel,
        mesh=plsc.VectorSubcoreMesh(core_axis_name="c", subcore_axis_name="s"),
        out_type=jax.ShapeDtypeStruct((NC, K, W), jnp.float32),
        scratch_types=[
            pltpu.VMEM((NCHUNK, CHUNK), jnp.int32),
            pltpu.VMEM((CHUNK, W), jnp.float32),
            pltpu.VMEM_SHARED((K, W), jnp.float32),
        ],
    )
    return run(zaug_half, idx3, jnp.zeros((K, W), jnp.float32))


def _finish_body(s1_ref, s2_ref, c_ref, w_ref, dc1_ref, dc2_ref, out_ref):
    both = s1_ref[0] + s1_ref[1] + s2_ref[0] + s2_ref[1]
    sums = both[:, :D]
    counts = both[:, D]
    means = sums / jnp.maximum(counts, 1.0)[:, None]
    diff = c_ref[...] - means
    cd = jnp.sqrt(jnp.sum(diff * diff, axis=1) + 1e-12)
    nonempty = (counts > 0.0).astype(jnp.float32)
    nw = nonempty * w_ref[0, :]
    nc = jnp.sum(nw * cd) / jnp.maximum(jnp.sum(nw), 1e-12)
    dc = (dc1_ref[0, 0] + dc2_ref[0, 0]) / N
    out_ref[...] = jnp.full((1, 1), nc + dc, jnp.float32)


def _finish_stage(sums1, sums2, centers, weights, dc1, dc2):
    return pl.pallas_call(
        _finish_body,
        out_shape=jax.ShapeDtypeStruct((1, 1), jnp.float32),
    )(sums1, sums2, centers, weights.reshape(1, K), dc1, dc2)


@jax.jit
def kernel(z, centers, weights):
    z1, z2 = z[:H], z[H:]
    assign1, dc1, zaug1 = _assign_stage(z1, centers)
    sums1 = _segsum_stage(zaug1, assign1.reshape(H))
    assign2, dc2, zaug2 = _assign_stage(z2, centers)
    sums2 = _segsum_stage(zaug2, assign2.reshape(H))
    out = _finish_stage(sums1, sums2, centers, weights, dc1, dc2)
    return out[0, 0]


# speedup vs baseline: 1.0669x; 1.0669x over previous
import jax
import jax.numpy as jnp
from jax import lax
from jax.experimental import pallas as pl
from jax.experimental.pallas import tpu as pltpu
from jax.experimental.pallas import tpu_sc as plsc

N = 16384
D = 64
K = 512
W = 128
BLK = 2048
GRID = N // BLK

NC = 2
NS = 16
NW = NC * NS
RPW = N // NW
CHUNK = 128
NCHUNK = RPW // CHUNK

_IDXMASK = 511


def _assign_body(z_ref, c_ref, assign_ref, dc_ref, zaug_ref, caug_ref, dcv_ref):
    i = pl.program_id(0)

    @pl.when(i == 0)
    def _init():
        c = c_ref[...]
        cc = jnp.sum(c * c, axis=1, keepdims=True)
        caug_ref[...] = jnp.concatenate(
            [c, jnp.ones((K, 1), jnp.float32), cc], axis=1)
        dcv_ref[...] = jnp.zeros_like(dcv_ref)

    z = z_ref[...]
    zz = jnp.sum(z * z, axis=1, keepdims=True)
    ones = jnp.ones((BLK, 1), jnp.float32)
    zaug_ref[...] = jnp.concatenate(
        [z, ones, jnp.zeros((BLK, W - D - 1), jnp.float32)], axis=1)
    d2 = jnp.maximum(jax.lax.dot_general(
        jnp.concatenate([-2.0 * z, zz, ones], axis=1), caug_ref[...],
        dimension_numbers=(((1,), (1,)), ((), ())),
        preferred_element_type=jnp.float32,
    ), 0.0)

    iota = jax.lax.broadcasted_iota(jnp.int32, (BLK, K), 1)
    key = (jax.lax.bitcast_convert_type(d2, jnp.int32) & ~_IDXMASK) | iota
    kmin = jnp.min(key, axis=1, keepdims=True)
    assign_ref[...] = kmin & _IDXMASK
    m = jax.lax.bitcast_convert_type(kmin & ~_IDXMASK, jnp.float32)
    dcv_ref[...] += jnp.sqrt(m + 1e-12)

    @pl.when(i == GRID - 1)
    def _finish():
        dc_ref[...] = jnp.sum(dcv_ref[...]).reshape(1, 1)


def _assign_stage(z, centers):
    return pl.pallas_call(
        _assign_body,
        grid=(GRID,),
        in_specs=[
            pl.BlockSpec((BLK, D), lambda i: (i, 0)),
            pl.BlockSpec((K, D), lambda i: (0, 0)),
        ],
        out_specs=[
            pl.BlockSpec((BLK, 1), lambda i: (i, 0)),
            pl.BlockSpec((1, 1), lambda i: (0, 0)),
            pl.BlockSpec((BLK, W), lambda i: (i, 0)),
        ],
        out_shape=[
            jax.ShapeDtypeStruct((N, 1), jnp.int32),
            jax.ShapeDtypeStruct((1, 1), jnp.float32),
            jax.ShapeDtypeStruct((N, W), jnp.float32),
        ],
        scratch_shapes=[
            pltpu.VMEM((K, D + 2), jnp.float32),
            pltpu.VMEM((BLK, 1), jnp.float32),
        ],
    )(z, centers)


def _segsum_kernel(zaug_hbm, idx_hbm, zeros_hbm, sums_hbm,
                   idx_v, zrows_v, acc_s):
    c = lax.axis_index("c")
    s = lax.axis_index("s")
    wid = s * NC + c

    pltpu.sync_copy(idx_hbm.at[wid], idx_v)

    @pl.when(s == 0)
    def _zero():
        pltpu.sync_copy(zeros_hbm, acc_s)

    plsc.subcore_barrier()

    for j in range(NCHUNK):
        pltpu.sync_copy(zaug_hbm.at[pl.ds(wid * RPW + j * CHUNK, CHUNK)],
                        zrows_v)
        pltpu.sync_copy(zrows_v, acc_s.at[idx_v.at[j]], add=True)

    plsc.subcore_barrier()

    @pl.when(s == 0)
    def _flush():
        pltpu.sync_copy(acc_s, sums_hbm.at[c])


def _segsum_stage(zaug, assign):
    idx3 = assign.reshape(NW, NCHUNK, CHUNK)
    run = pl.kernel(
        _segsum_kernel,
        mesh=plsc.VectorSubcoreMesh(core_axis_name="c", subcore_axis_name="s"),
        out_type=jax.ShapeDtypeStruct((NC, K, W), jnp.float32),
        scratch_types=[
            pltpu.VMEM((NCHUNK, CHUNK), jnp.int32),
            pltpu.VMEM((CHUNK, W), jnp.float32),
            pltpu.VMEM_SHARED((K, W), jnp.float32),
        ],
    )
    return run(zaug, idx3, jnp.zeros((K, W), jnp.float32))


def _finish_body(sums_ref, c_ref, w_ref, dc_ref, out_ref):
    both = sums_ref[0] + sums_ref[1]
    sums = both[:, :D]
    counts = both[:, D]
    means = sums / jnp.maximum(counts, 1.0)[:, None]
    diff = c_ref[...] - means
    cd = jnp.sqrt(jnp.sum(diff * diff, axis=1) + 1e-12)
    nonempty = (counts > 0.0).astype(jnp.float32)
    nw = nonempty * w_ref[0, :]
    nc = jnp.sum(nw * cd) / jnp.maximum(jnp.sum(nw), 1e-12)
    out_ref[...] = jnp.full((1, 1), nc + dc_ref[0, 0] / N, jnp.float32)


def _finish_stage(sums_p, centers, weights, dc):
    return pl.pallas_call(
        _finish_body,
        out_shape=jax.ShapeDtypeStruct((1, 1), jnp.float32),
    )(sums_p, centers, weights.reshape(1, K), dc)


@jax.jit
def kernel(z, centers, weights):
    assign, dc, zaug = _assign_stage(z, centers)
    sums_p = _segsum_stage(zaug, assign.reshape(N))
    out = _finish_stage(sums_p, centers, weights, dc)
    return out[0, 0]


# speedup vs baseline: 1.1272x; 1.0565x over previous
import jax
import jax.numpy as jnp
from jax import lax
from jax.experimental import pallas as pl
from jax.experimental.pallas import tpu as pltpu
from jax.experimental.pallas import tpu_sc as plsc

N = 16384
D = 64
K = 512
W = 128
BLK = 2048
GRID = N // BLK

NC = 2
NS = 16
NW = NC * NS
RPW = N // NW
CHUNK = 128
NCHUNK = RPW // CHUNK

_IDXMASK = 511


def _assign_body(z_ref, c_ref, assign_ref, dc_ref, zaug_ref, caug_ref, dcv_ref):
    i = pl.program_id(0)

    @pl.when(i == 0)
    def _init():
        c = c_ref[...]
        cc = jnp.sum(c * c, axis=1, keepdims=True)
        caug_ref[...] = jnp.concatenate(
            [c, jnp.ones((K, 1), jnp.float32), cc], axis=1)
        dcv_ref[...] = jnp.zeros_like(dcv_ref)

    z = z_ref[...]
    zz = jnp.sum(z * z, axis=1, keepdims=True)
    ones = jnp.ones((BLK, 1), jnp.float32)
    zaug_ref[...] = jnp.concatenate(
        [z, ones, jnp.zeros((BLK, W - D - 1), jnp.float32)], axis=1)
    d2 = jnp.maximum(jax.lax.dot_general(
        jnp.concatenate([-2.0 * z, zz, ones], axis=1), caug_ref[...],
        dimension_numbers=(((1,), (1,)), ((), ())),
        preferred_element_type=jnp.float32,
    ), 0.0)

    iota = jax.lax.broadcasted_iota(jnp.int32, (BLK, K), 1)
    key = (jax.lax.bitcast_convert_type(d2, jnp.int32) & ~_IDXMASK) | iota
    kmin = jnp.min(key, axis=1, keepdims=True)
    assign_ref[...] = kmin & _IDXMASK
    m = jax.lax.bitcast_convert_type(kmin & ~_IDXMASK, jnp.float32)
    dcv_ref[...] += jnp.sqrt(m + 1e-12)

    @pl.when(i == GRID - 1)
    def _finish():
        dc_ref[...] = jnp.sum(dcv_ref[...]).reshape(1, 1)


def _assign_stage(z, centers):
    return pl.pallas_call(
        _assign_body,
        grid=(GRID,),
        in_specs=[
            pl.BlockSpec((BLK, D), lambda i: (i, 0)),
            pl.BlockSpec((K, D), lambda i: (0, 0)),
        ],
        out_specs=[
            pl.BlockSpec((BLK, 1), lambda i: (i, 0)),
            pl.BlockSpec((1, 1), lambda i: (0, 0)),
            pl.BlockSpec((BLK, W), lambda i: (i, 0)),
        ],
        out_shape=[
            jax.ShapeDtypeStruct((N, 1), jnp.int32),
            jax.ShapeDtypeStruct((1, 1), jnp.float32),
            jax.ShapeDtypeStruct((N, W), jnp.float32),
        ],
        scratch_shapes=[
            pltpu.VMEM((K, D + 2), jnp.float32),
            pltpu.VMEM((BLK, 1), jnp.float32),
        ],
    )(z, centers)


def _segsum_kernel(zaug_hbm, idx_hbm, zeros_hbm, sums_hbm,
                   idx_v, zr0, zr1, acc_s, sem0, sem1):
    c = lax.axis_index("c")
    s = lax.axis_index("s")
    wid = s * NC + c

    pltpu.sync_copy(idx_hbm.at[wid], idx_v)

    bufs = (zr0, zr1)
    sems = (sem0, sem1)
    pending = pltpu.async_copy(
        zaug_hbm.at[pl.ds(wid * RPW, CHUNK)], zr0, sem0)

    @pl.when(s == 0)
    def _zero():
        pltpu.sync_copy(zeros_hbm, acc_s)

    plsc.subcore_barrier()

    for j in range(NCHUNK):
        nxt = None
        if j + 1 < NCHUNK:
            nxt = pltpu.async_copy(
                zaug_hbm.at[pl.ds(wid * RPW + (j + 1) * CHUNK, CHUNK)],
                bufs[(j + 1) % 2], sems[(j + 1) % 2])
        pending.wait()
        pltpu.sync_copy(bufs[j % 2], acc_s.at[idx_v.at[j]], add=True)
        pending = nxt

    plsc.subcore_barrier()

    @pl.when(s == 0)
    def _flush():
        pltpu.sync_copy(acc_s, sums_hbm.at[c])


def _segsum_stage(zaug, assign):
    idx3 = assign.reshape(NW, NCHUNK, CHUNK)
    run = pl.kernel(
        _segsum_kernel,
        mesh=plsc.VectorSubcoreMesh(core_axis_name="c", subcore_axis_name="s"),
        out_type=jax.ShapeDtypeStruct((NC, K, W), jnp.float32),
        scratch_types=[
            pltpu.VMEM((NCHUNK, CHUNK), jnp.int32),
            pltpu.VMEM((CHUNK, W), jnp.float32),
            pltpu.VMEM((CHUNK, W), jnp.float32),
            pltpu.VMEM_SHARED((K, W), jnp.float32),
            pltpu.SemaphoreType.DMA,
            pltpu.SemaphoreType.DMA,
        ],
    )
    return run(zaug, idx3, jnp.zeros((K, W), jnp.float32))


def _finish_body(sums_ref, c_ref, w_ref, dc_ref, out_ref):
    both = sums_ref[0] + sums_ref[1]
    sums = both[:, :D]
    counts = both[:, D]
    means = sums / jnp.maximum(counts, 1.0)[:, None]
    diff = c_ref[...] - means
    cd = jnp.sqrt(jnp.sum(diff * diff, axis=1) + 1e-12)
    nonempty = (counts > 0.0).astype(jnp.float32)
    nw = nonempty * w_ref[0, :]
    nc = jnp.sum(nw * cd) / jnp.maximum(jnp.sum(nw), 1e-12)
    out_ref[...] = jnp.full((1, 1), nc + dc_ref[0, 0] / N, jnp.float32)


def _finish_stage(sums_p, centers, weights, dc):
    return pl.pallas_call(
        _finish_body,
        out_shape=jax.ShapeDtypeStruct((1, 1), jnp.float32),
    )(sums_p, centers, weights.reshape(1, K), dc)


@jax.jit
def kernel(z, centers, weights):
    assign, dc, zaug = _assign_stage(z, centers)
    sums_p = _segsum_stage(zaug, assign.reshape(N))
    out = _finish_stage(sums_p, centers, weights, dc)
    return out[0, 0]
